# baseline (speedup 1.0000x reference)
import functools

import jax
import jax.numpy as jnp
from jax import lax
from jax.experimental import pallas as pl
from jax.experimental.pallas import tpu as pltpu
from jax.experimental.pallas import tpu_sc as plsc

_VOCAB = 18440
_EMBED = 100
_OUT = 2
_B = 4096
_L = 200

_NW = 32
_RPW = _B // _NW
_GPW = _RPW // 16
_LW = _L // 2


def _tw_body(wt_ref, t_ref, o_ref):
    tw_t = lax.dot_general(
        wt_ref[...], t_ref[...],
        (((1,), (1,)), ((), ())),
        preferred_element_type=jnp.float32,
    ) * (1.0 / _L)
    bits = lax.bitcast_convert_type(tw_t.astype(jnp.bfloat16), jnp.uint16)
    packed = bits[0].astype(jnp.uint32) | (bits[1].astype(jnp.uint32) << 16)
    o_ref[...] = packed.astype(jnp.int32)


def _fold_table(table, Wt):
    return pl.pallas_call(
        _tw_body,
        out_shape=jax.ShapeDtypeStruct((_VOCAB,), jnp.int32),
    )(Wt, table)


@functools.partial(
    pl.kernel,
    out_type=jax.ShapeDtypeStruct((_NW, _OUT, _RPW), jnp.float32),
    mesh=plsc.VectorSubcoreMesh(core_axis_name="c", subcore_axis_name="s"),
    compiler_params=pltpu.CompilerParams(needs_layout_passes=False),
    scratch_types=[
        pltpu.VMEM((_RPW * _LW,), jnp.int32),
        pltpu.VMEM((_VOCAB,), jnp.int32),
        pltpu.VMEM((_OUT * 16,), jnp.float32),
        pltpu.VMEM((_OUT, _RPW), jnp.float32),
    ],
)
def _sc_pool(tok_hbm, tw_hbm, bias_hbm, out_hbm, tok_v, tw_v, bias_v, out_v):
    wid = lax.axis_index("s") * 2 + lax.axis_index("c")
    nword = _RPW * _LW
    pltpu.sync_copy(tok_hbm.at[pl.ds(wid * nword, nword)], tok_v)
    pltpu.sync_copy(tw_hbm, tw_v)
    pltpu.sync_copy(bias_hbm, bias_v)
    bv0 = bias_v[pl.ds(0, 16)]
    bv1 = bias_v[pl.ds(16, 16)]
    row_stride = lax.broadcasted_iota(jnp.int32, (16,), 0) * _LW
    lo_mask = jnp.full((16,), 0xFFFF, jnp.int32)
    hi_mask = jnp.full((16,), 0xFFFF0000, jnp.uint32).astype(jnp.int32)

    def unpack_pair(w):
        v0 = lax.bitcast_convert_type(lax.shift_left(w, 16), jnp.float32)
        v1 = lax.bitcast_convert_type(lax.bitwise_and(w, hi_mask), jnp.float32)
        return v0, v1

    def body(j, carry):
        new = []
        for g in range(_GPW):
            a0, a1 = carry[2 * g], carry[2 * g + 1]
            tokw = plsc.load_gather(tok_v, [row_stride + (g * (16 * _LW) + j)])
            t_lo = lax.bitwise_and(tokw, lo_mask)
            t_hi = lax.shift_right_logical(tokw, 16)
            w_lo = plsc.load_gather(tw_v, [t_lo])
            w_hi = plsc.load_gather(tw_v, [t_hi])
            v0l, v1l = unpack_pair(w_lo)
            v0h, v1h = unpack_pair(w_hi)
            new.append(a0 + (v0l + v0h))
            new.append(a1 + (v1l + v1h))
        return tuple(new)

    zero = jnp.zeros((16,), jnp.float32)
    accs = lax.fori_loop(0, _LW, body, (zero,) * (2 * _GPW))
    for g in range(_GPW):
        out_v[0, pl.ds(g * 16, 16)] = accs[2 * g] + bv0
        out_v[1, pl.ds(g * 16, 16)] = accs[2 * g + 1] + bv1

    pltpu.sync_copy(out_v, out_hbm.at[wid])


def kernel(text_token, table, W, b):
    tw_pack = _fold_table(table, W.T)
    tok_words = lax.bitcast_convert_type(
        text_token.astype(jnp.int16).reshape(_B * _LW, 2), jnp.int32
    )
    bias16 = jnp.broadcast_to(b[:, None], (_OUT, 16)).reshape(-1)
    out = _sc_pool(tok_words, tw_pack, bias16)
    return jnp.transpose(out, (0, 2, 1)).reshape(_B, _OUT)


# speedup vs baseline: 3.7604x; 3.7604x over previous
import functools

import jax
import jax.numpy as jnp
from jax import lax
from jax.experimental import pallas as pl
from jax.experimental.pallas import tpu as pltpu
from jax.experimental.pallas import tpu_sc as plsc

_VOCAB = 18440
_EMBED = 100
_OUT = 2
_B = 4096
_L = 200

_NW = 32
_RPW = _B // _NW
_GPW = _RPW // 16
_LW = _L // 2
_TP = 128


def _prep_body(wt_ref, t_ref, tok_ref, tw_ref, tokw_ref):
    tw_t = lax.dot_general(
        wt_ref[...], t_ref[...],
        (((1,), (1,)), ((), ())),
        preferred_element_type=jnp.float32,
    ) * (1.0 / _L)
    bits = lax.bitcast_convert_type(tw_t.astype(jnp.bfloat16), jnp.uint16)
    packed = bits[0].astype(jnp.uint32) | (bits[1].astype(jnp.uint32) << 16)
    tw_ref[...] = packed.astype(jnp.int32)

    tok = tok_ref[...]
    words = tok[:, :_LW] | (tok[:, _LW:] << 16)
    tokw_ref[:, : _LW] = words


def _prep(table, Wt, text_token):
    return pl.pallas_call(
        _prep_body,
        out_shape=(
            jax.ShapeDtypeStruct((_VOCAB,), jnp.int32),
            jax.ShapeDtypeStruct((_B, _TP), jnp.int32),
        ),
    )(Wt, table, text_token)


@functools.partial(
    pl.kernel,
    out_type=jax.ShapeDtypeStruct((_NW, _OUT, _RPW), jnp.float32),
    mesh=plsc.VectorSubcoreMesh(core_axis_name="c", subcore_axis_name="s"),
    compiler_params=pltpu.CompilerParams(needs_layout_passes=False),
    scratch_types=[
        pltpu.VMEM((_RPW * _TP,), jnp.int32),
        pltpu.VMEM((_VOCAB,), jnp.int32),
        pltpu.VMEM((_OUT * 16,), jnp.float32),
        pltpu.VMEM((_OUT, _RPW), jnp.float32),
    ],
)
def _sc_pool(tok_hbm, tw_hbm, bias_hbm, out_hbm, tok_v, tw_v, bias_v, out_v):
    wid = lax.axis_index("s") * 2 + lax.axis_index("c")
    nword = _RPW * _TP
    pltpu.sync_copy(tok_hbm.at[pl.ds(wid * nword, nword)], tok_v)
    pltpu.sync_copy(tw_hbm, tw_v)
    pltpu.sync_copy(bias_hbm, bias_v)
    bv0 = bias_v[pl.ds(0, 16)]
    bv1 = bias_v[pl.ds(16, 16)]
    row_stride = lax.broadcasted_iota(jnp.int32, (16,), 0) * _TP
    lo_mask = jnp.full((16,), 0xFFFF, jnp.int32)
    hi_mask = jnp.full((16,), 0xFFFF0000, jnp.uint32).astype(jnp.int32)

    def unpack_pair(w):
        v0 = lax.bitcast_convert_type(lax.shift_left(w, 16), jnp.float32)
        v1 = lax.bitcast_convert_type(lax.bitwise_and(w, hi_mask), jnp.float32)
        return v0, v1

    def body(j, carry):
        new = []
        for g in range(_GPW):
            a0, a1 = carry[2 * g], carry[2 * g + 1]
            tokw = plsc.load_gather(tok_v, [row_stride + (g * (16 * _TP) + j)])
            t_lo = lax.bitwise_and(tokw, lo_mask)
            t_hi = lax.shift_right_logical(tokw, 16)
            w_lo = plsc.load_gather(tw_v, [t_lo])
            w_hi = plsc.load_gather(tw_v, [t_hi])
            v0l, v1l = unpack_pair(w_lo)
            v0h, v1h = unpack_pair(w_hi)
            new.append(a0 + (v0l + v0h))
            new.append(a1 + (v1l + v1h))
        return tuple(new)

    zero = jnp.zeros((16,), jnp.float32)
    accs = lax.fori_loop(0, _LW, body, (zero,) * (2 * _GPW))
    for g in range(_GPW):
        out_v[0, pl.ds(g * 16, 16)] = accs[2 * g] + bv0
        out_v[1, pl.ds(g * 16, 16)] = accs[2 * g + 1] + bv1

    pltpu.sync_copy(out_v, out_hbm.at[wid])


def kernel(text_token, table, W, b):
    tw_pack, tok_words = _prep(table, W.T, text_token)
    bias16 = jnp.broadcast_to(b[:, None], (_OUT, 16)).reshape(-1)
    out = _sc_pool(tok_words.reshape(-1), tw_pack, bias16)
    return jnp.transpose(out, (0, 2, 1)).reshape(_B, _OUT)


# speedup vs baseline: 3.7687x; 1.0022x over previous
import functools

import jax
import jax.numpy as jnp
from jax import lax
from jax.experimental import pallas as pl
from jax.experimental.pallas import tpu as pltpu
from jax.experimental.pallas import tpu_sc as plsc

_VOCAB = 18440
_EMBED = 100
_OUT = 2
_B = 4096
_L = 200

_NW = 32
_RPW = _B // _NW
_GPW = _RPW // 16
_LW = _L // 2
_TP = 128


def _prep_body(wt_ref, t_ref, tok_ref, tw_ref, tokw_ref):
    tw_t = lax.dot_general(
        wt_ref[...], t_ref[...],
        (((1,), (1,)), ((), ())),
        preferred_element_type=jnp.float32,
    ) * (1.0 / _L)
    bits = lax.bitcast_convert_type(tw_t.astype(jnp.bfloat16), jnp.uint16)
    packed = bits[0].astype(jnp.uint32) | (bits[1].astype(jnp.uint32) * 65536)
    tw_ref[...] = packed.astype(jnp.int32)

    tok = tok_ref[...]
    tokw_ref[:, : _LW] = tok[:, : _LW] | (tok[:, _LW :] * 65536)


def _prep(table, Wt, text_token):
    return pl.pallas_call(
        _prep_body,
        out_shape=(
            jax.ShapeDtypeStruct((_VOCAB,), jnp.int32),
            jax.ShapeDtypeStruct((_B, _TP), jnp.int32),
        ),
    )(Wt, table, text_token)


@functools.partial(
    pl.kernel,
    out_type=jax.ShapeDtypeStruct((_NW, _OUT, _RPW), jnp.float32),
    mesh=plsc.VectorSubcoreMesh(core_axis_name="c", subcore_axis_name="s"),
    compiler_params=pltpu.CompilerParams(needs_layout_passes=False),
    scratch_types=[
        pltpu.VMEM((_RPW * _TP,), jnp.int32),
        pltpu.VMEM((_VOCAB,), jnp.int32),
        pltpu.VMEM((_OUT * 16,), jnp.float32),
        pltpu.VMEM((_OUT, _RPW), jnp.float32),
    ],
)
def _sc_pool(tok_hbm, tw_hbm, bias_hbm, out_hbm, tok_v, tw_v, bias_v, out_v):
    wid = lax.axis_index("s") * 2 + lax.axis_index("c")
    nword = _RPW * _TP
    pltpu.sync_copy(tok_hbm.at[pl.ds(wid * nword, nword)], tok_v)
    pltpu.sync_copy(tw_hbm, tw_v)
    pltpu.sync_copy(bias_hbm, bias_v)
    bv0 = bias_v[pl.ds(0, 16)]
    bv1 = bias_v[pl.ds(16, 16)]
    row_stride = lax.broadcasted_iota(jnp.int32, (16,), 0) * _TP
    lo_mask = jnp.full((16,), 0xFFFF, jnp.int32)
    hi_mask = jnp.full((16,), 0xFFFF0000, jnp.uint32).astype(jnp.int32)

    def unpack_pair(w):
        v0 = lax.bitcast_convert_type(lax.shift_left(w, 16), jnp.float32)
        v1 = lax.bitcast_convert_type(lax.bitwise_and(w, hi_mask), jnp.float32)
        return v0, v1

    def body(j, carry):
        new = []
        for g in range(_GPW):
            a0, a1 = carry[2 * g], carry[2 * g + 1]
            tokw = plsc.load_gather(tok_v, [row_stride + (g * (16 * _TP) + j)])
            t_lo = lax.bitwise_and(tokw, lo_mask)
            t_hi = lax.shift_right_logical(tokw, 16)
            w_lo = plsc.load_gather(tw_v, [t_lo])
            w_hi = plsc.load_gather(tw_v, [t_hi])
            v0l, v1l = unpack_pair(w_lo)
            v0h, v1h = unpack_pair(w_hi)
            new.append(a0 + (v0l + v0h))
            new.append(a1 + (v1l + v1h))
        return tuple(new)

    zero = jnp.zeros((16,), jnp.float32)
    accs = lax.fori_loop(0, _LW, body, (zero,) * (2 * _GPW))
    for g in range(_GPW):
        out_v[0, pl.ds(g * 16, 16)] = accs[2 * g] + bv0
        out_v[1, pl.ds(g * 16, 16)] = accs[2 * g + 1] + bv1

    pltpu.sync_copy(out_v, out_hbm.at[wid])


def kernel(text_token, table, W, b):
    tw_pack, tok_words = _prep(table, W.T, text_token)
    bias16 = jnp.broadcast_to(b[:, None], (_OUT, 16)).reshape(-1)
    out = _sc_pool(tok_words.reshape(-1), tw_pack, bias16)
    return jnp.transpose(out, (0, 2, 1)).reshape(_B, _OUT)


# speedup vs baseline: 4.3115x; 1.1440x over previous
import functools

import jax
import jax.numpy as jnp
from jax import lax
from jax.experimental import pallas as pl
from jax.experimental.pallas import tpu as pltpu
from jax.experimental.pallas import tpu_sc as plsc

_VOCAB = 18440
_EMBED = 100
_OUT = 2
_B = 4096
_L = 200

_NW = 32
_RPW = _B // _NW
_GPW = _RPW // 16
_LW = _L // 2
_TPJ = 104


def _prep_body(wt_ref, t_ref, tok_ref, tw_ref, tokw_ref):
    tw_t = lax.dot_general(
        wt_ref[...], t_ref[...],
        (((1,), (1,)), ((), ())),
        preferred_element_type=jnp.float32,
    ) * (1.0 / _L)
    bits = lax.bitcast_convert_type(tw_t.astype(jnp.bfloat16), jnp.uint16)
    packed = bits[0].astype(jnp.uint32) | (bits[1].astype(jnp.uint32) * 65536)
    tw_ref[...] = packed.astype(jnp.int32)

    tok = tok_ref[...]
    words = tok[:, : _LW] | (tok[:, _LW :] * 65536)
    w3 = lax.transpose(words.reshape(_NW, _RPW, _LW), (0, 2, 1))
    tokw_ref[:, : _LW, :] = w3


def _prep(table, Wt, text_token):
    return pl.pallas_call(
        _prep_body,
        out_shape=(
            jax.ShapeDtypeStruct((_VOCAB,), jnp.int32),
            jax.ShapeDtypeStruct((_NW, _TPJ, _RPW), jnp.int32),
        ),
    )(Wt, table, text_token)


@functools.partial(
    pl.kernel,
    out_type=jax.ShapeDtypeStruct((_NW, _OUT, _RPW), jnp.float32),
    mesh=plsc.VectorSubcoreMesh(core_axis_name="c", subcore_axis_name="s"),
    compiler_params=pltpu.CompilerParams(needs_layout_passes=False),
    scratch_types=[
        pltpu.VMEM((_TPJ * _RPW,), jnp.int32),
        pltpu.VMEM((_VOCAB,), jnp.int32),
        pltpu.VMEM((_OUT * 16,), jnp.float32),
        pltpu.VMEM((_OUT, _RPW), jnp.float32),
    ],
)
def _sc_pool(tok_hbm, tw_hbm, bias_hbm, out_hbm, tok_v, tw_v, bias_v, out_v):
    wid = lax.axis_index("s") * 2 + lax.axis_index("c")
    nword = _TPJ * _RPW
    pltpu.sync_copy(tok_hbm.at[pl.ds(wid * nword, nword)], tok_v)
    pltpu.sync_copy(tw_hbm, tw_v)
    pltpu.sync_copy(bias_hbm, bias_v)
    bv0 = bias_v[pl.ds(0, 16)]
    bv1 = bias_v[pl.ds(16, 16)]
    lo_mask = jnp.full((16,), 0xFFFF, jnp.int32)
    hi_mask = jnp.full((16,), 0xFFFF0000, jnp.uint32).astype(jnp.int32)

    def unpack_pair(w):
        v0 = lax.bitcast_convert_type(lax.shift_left(w, 16), jnp.float32)
        v1 = lax.bitcast_convert_type(lax.bitwise_and(w, hi_mask), jnp.float32)
        return v0, v1

    def body(j, carry):
        new = []
        for g in range(_GPW):
            a0, a1 = carry[2 * g], carry[2 * g + 1]
            tokw = tok_v[pl.ds(j * _RPW + g * 16, 16)]
            t_lo = lax.bitwise_and(tokw, lo_mask)
            t_hi = lax.shift_right_logical(tokw, 16)
            w_lo = plsc.load_gather(tw_v, [t_lo])
            w_hi = plsc.load_gather(tw_v, [t_hi])
            v0l, v1l = unpack_pair(w_lo)
            v0h, v1h = unpack_pair(w_hi)
            new.append(a0 + (v0l + v0h))
            new.append(a1 + (v1l + v1h))
        return tuple(new)

    zero = jnp.zeros((16,), jnp.float32)
    accs = lax.fori_loop(0, _LW, body, (zero,) * (2 * _GPW))
    for g in range(_GPW):
        out_v[0, pl.ds(g * 16, 16)] = accs[2 * g] + bv0
        out_v[1, pl.ds(g * 16, 16)] = accs[2 * g + 1] + bv1

    pltpu.sync_copy(out_v, out_hbm.at[wid])


def kernel(text_token, table, W, b):
    tw_pack, tok_words = _prep(table, W.T, text_token)
    bias16 = jnp.broadcast_to(b[:, None], (_OUT, 16)).reshape(-1)
    out = _sc_pool(tok_words.reshape(-1), tw_pack, bias16)
    return jnp.transpose(out, (0, 2, 1)).reshape(_B, _OUT)
